# baseline (speedup 1.0000x reference)
import functools

import jax
import jax.numpy as jnp
from jax import lax
from jax.experimental import pallas as pl
from jax.experimental.pallas import tpu as pltpu
from jax.experimental.pallas import tpu_sc as plsc

NUM_VOCAB = 1000000
D = 64
PD = 128
BATCH = 4096
SEQ = 200
LANES = 16
NL = 13

NC = 2
NS = 16
NW = NC * NS

SEQ_PER_W = BATCH // NW
NB = 2


def _emb_body(x_hbm, emb_hbm, pos_hbm, out_hbm,
              xb_v, idx0_v, idx1_v, rows_v, outb_v, pos_s, gsem, osem):
    wid = lax.axis_index("s") * NC + lax.axis_index("c")
    col = wid * SEQ_PER_W

    pltpu.sync_copy(x_hbm.at[:, pl.ds(col, SEQ_PER_W)], xb_v)
    pltpu.sync_copy(pos_hbm, pos_s)
    iota = lax.iota(jnp.int32, LANES)
    c199 = jnp.full((LANES,), 199, jnp.int32)

    def gather_start(g, b):
        idxb = idx0_v if b == 0 else idx1_v
        bvec = jnp.full((LANES,), 0, jnp.int32) + g
        for l0 in range(NL):
            lvec = iota + (l0 * LANES)
            lc = jnp.minimum(lvec, c199)
            xv = plsc.load_gather(xb_v, [lc, bvec])
            xq = lax.shift_right_logical(xv, 1)
            if l0 == NL - 1:
                plsc.store_scatter(idxb, [lvec], xq, mask=lvec < SEQ)
            else:
                plsc.store_scatter(idxb, [lvec], xq)
        pltpu.async_copy(emb_hbm.at[idxb], rows_v.at[b], gsem.at[b])

    def gather_wait(g, b):
        idxb = idx0_v if b == 0 else idx1_v
        pltpu.make_async_copy(emb_hbm.at[idxb], rows_v.at[b],
                              gsem.at[b]).wait()

    def out_start(g):
        pltpu.async_copy(outb_v, out_hbm.at[pl.ds(col + g, 1)], osem)

    def out_wait(g):
        pltpu.make_async_copy(outb_v, out_hbm.at[pl.ds(col + g, 1)],
                              osem).wait()

    def compute(g, b):
        rows = rows_v.at[b]
        bvec = jnp.full((LANES,), 0, jnp.int32) + g

        def r_body(r, _):
            rvec = jnp.full((LANES,), 0, jnp.int32) + r
            zvec = jnp.full((LANES,), 0, jnp.int32)
            pbase = jnp.full((LANES,), 0, jnp.int32) + r * D
            xv = plsc.load_gather(xb_v, [rvec, bvec])
            half = lax.shift_left(jnp.bitwise_and(xv, 1), 6)
            for j in range(D // LANES):
                cvec = iota + (j * LANES)
                v = plsc.load_gather(rows, [rvec, half + cvec])
                p = plsc.load_gather(pos_s, [pbase + cvec])
                plsc.store_scatter(outb_v, [zvec, rvec, cvec], v + p)
            return _
        lax.fori_loop(0, SEQ, r_body, 0)

    for g in range(NB - 1):
        gather_start(g, g)

    def seq_body(grp, carry):
        for u in range(NB):
            g = grp * NB + u
            b = u

            @pl.when(g + NB - 1 < SEQ_PER_W)
            def _():
                gather_start(g + NB - 1, (u + NB - 1) % NB)

            gather_wait(g, b)

            @pl.when(g >= 1)
            def _():
                out_wait(g - 1)

            compute(g, b)
            out_start(g)
        return carry

    lax.fori_loop(0, SEQ_PER_W // NB, seq_body, 0)
    out_wait(SEQ_PER_W - 1)


@jax.jit
def kernel(x, emb_table, pos_table):
    x_t = x.T.astype(jnp.int32)
    emb2 = emb_table.reshape(NUM_VOCAB // 2, 2 * D)
    pos_flat = pos_table[:SEQ].reshape(-1)

    mesh = plsc.VectorSubcoreMesh(core_axis_name="c", subcore_axis_name="s")
    run = pl.kernel(
        _emb_body,
        mesh=mesh,
        out_type=jax.ShapeDtypeStruct((BATCH, SEQ, D), jnp.float32),
        compiler_params=pltpu.CompilerParams(needs_layout_passes=False),
        scratch_types=[
            pltpu.VMEM((SEQ, SEQ_PER_W), jnp.int32),
            pltpu.VMEM((SEQ,), jnp.int32),
            pltpu.VMEM((SEQ,), jnp.int32),
            pltpu.VMEM((NB, SEQ, PD), jnp.float32),
            pltpu.VMEM((1, SEQ, D), jnp.float32),
            pltpu.VMEM((SEQ * D,), jnp.float32),
            pltpu.SemaphoreType.DMA((NB,)),
            pltpu.SemaphoreType.DMA,
        ],
    )
    return run(x_t, emb2, pos_flat)


# speedup vs baseline: 1.4429x; 1.4429x over previous
import functools

import jax
import jax.numpy as jnp
from jax import lax
from jax.experimental import pallas as pl
from jax.experimental.pallas import tpu as pltpu
from jax.experimental.pallas import tpu_sc as plsc

NUM_VOCAB = 1000000
MAXLEN = 200
D = 64
BATCH = 4096
SEQ = 200

NC = 2
NS = 16
NW = NC * NS

B_TOTAL = BATCH * SEQ
ROWS_PER_W = B_TOTAL // NW
CH = SEQ
NSTEP = ROWS_PER_W // CH
NBUF = 4
SEQ_PER_W = BATCH // NW


def _emb_body(x_hbm, emb_hbm, pos_hbm, out_hbm,
              idx_v, pos_v, rows_v, gsem, osem):
    wid = lax.axis_index("s") * NC + lax.axis_index("c")
    my_base = wid * ROWS_PER_W
    my_seq = wid * SEQ_PER_W

    pltpu.sync_copy(x_hbm.at[pl.ds(my_base, ROWS_PER_W)], idx_v)
    pltpu.sync_copy(pos_hbm.at[pl.ds(0, SEQ)], pos_v)

    def gather_start(g, b):
        idx = idx_v.at[pl.ds(g * CH, CH)]
        pltpu.async_copy(emb_hbm.at[idx], rows_v.at[b], gsem.at[b])

    def gather_wait(g, b):
        idx = idx_v.at[pl.ds(g * CH, CH)]
        pltpu.make_async_copy(emb_hbm.at[idx], rows_v.at[b], gsem.at[b]).wait()

    def out_start(g, b):
        pltpu.async_copy(rows_v.at[b], out_hbm.at[my_seq + g], osem.at[b])

    def out_wait(g, b):
        pltpu.make_async_copy(rows_v.at[b], out_hbm.at[my_seq + g],
                              osem.at[b]).wait()

    for b in range(NBUF - 1):
        gather_start(b, b)

    def group_body(grp, carry):
        for b in range(NBUF):
            g = grp * NBUF + b
            gather_wait(g, b)

            def add_row(r, c):
                for j in range(D // 16):
                    sl = pl.ds(j * 16, 16)
                    rows_v[b, r, sl] = rows_v[b, r, sl] + pos_v[r, sl]
                return c
            lax.fori_loop(0, CH, add_row, 0)

            out_start(g, b)

            b2 = (g + NBUF - 1) % NBUF

            @pl.when(g + NBUF - 1 < NSTEP)
            def _():
                @pl.when(g > 0)
                def _():
                    out_wait(g - 1, b2)
                gather_start(g + NBUF - 1, b2)
        return carry

    lax.fori_loop(0, NSTEP // NBUF, group_body, 0)

    for b in range(NBUF):
        g = NSTEP - NBUF + b
        out_wait(g, b)


@jax.jit
def kernel(x, emb_table, pos_table):
    x_flat = x.reshape(-1).astype(jnp.int32)

    mesh = plsc.VectorSubcoreMesh(core_axis_name="c", subcore_axis_name="s")
    run = pl.kernel(
        _emb_body,
        mesh=mesh,
        out_type=jax.ShapeDtypeStruct((BATCH, SEQ, D), jnp.float32),
        compiler_params=pltpu.CompilerParams(use_tc_tiling_on_sc=False),
        scratch_types=[
            pltpu.VMEM((ROWS_PER_W,), jnp.int32),
            pltpu.VMEM((SEQ, D), jnp.float32),
            pltpu.VMEM((NBUF, CH, D), jnp.float32),
            pltpu.SemaphoreType.DMA((NBUF,)),
            pltpu.SemaphoreType.DMA((NBUF,)),
        ],
    )
    return run(x_flat, emb_table, pos_table)
